# baseline (speedup 1.0000x reference)
import functools

import jax
import jax.numpy as jnp
from jax import lax
from jax.experimental import pallas as pl
from jax.experimental.pallas import tpu as pltpu
from jax.experimental.pallas import tpu_sc as plsc

_SC_CHUNK = 128


def _sin(x):
    ni = jnp.round(x * 0.3183098861837907).astype(jnp.int32)
    n = ni.astype(jnp.float32)
    r = x - n * 3.140625
    r = r - n * 9.676535897932e-4
    r2 = r * r
    p = r * (1.0 + r2 * (-0.16666666666 + r2 * (8.3333333333e-3 + r2 * (
        -1.98412698e-4 + r2 * (2.75573192e-6 + r2 * -2.50521084e-8)))))
    return jnp.where((ni & 1) == 1, -p, p)


def _sc_gather(word_table, ids):
    n = ids.shape[0]
    h = word_table.shape[1]
    info = plsc.get_sparse_core_info()
    nw = info.num_cores * info.num_subcores
    per = n // nw
    ch = min(_SC_CHUNK, per)
    nch = per // ch

    mesh = plsc.VectorSubcoreMesh(core_axis_name="c", subcore_axis_name="s")

    @functools.partial(
        pl.kernel,
        mesh=mesh,
        out_type=jax.ShapeDtypeStruct((n, h), jnp.float32),
        scratch_types=[
            pltpu.VMEM((ch,), jnp.int32),
            pltpu.VMEM((ch, h), jnp.float32),
            pltpu.SemaphoreType.DMA,
        ],
    )
    def gather_kernel(wt_hbm, ids_hbm, out_w, idx_v, rows_v, sem):
        wid = lax.axis_index("s") * info.num_cores + lax.axis_index("c")
        base = wid * per
        for c in range(nch):
            off = base + c * ch
            pltpu.sync_copy(ids_hbm.at[pl.ds(off, ch)], idx_v)
            pltpu.async_copy(wt_hbm.at[idx_v], rows_v, sem).wait()
            pltpu.sync_copy(rows_v, out_w.at[pl.ds(off, ch)])

    return gather_kernel(word_table, ids)


def _tc_body(*refs, ntt, nvs, ncat, eps):
    (wt_ref, ts_ref, tsp_ref, age_ref, tt_ref, vs_ref, vo_ref,
     ww_ref, wta_ref, tw_ref, tph_ref, aw_ref, aph_ref,
     cat_ref, b_ref, g_ref, bb_ref, o_ref) = refs[-18:]
    delta = ts_ref[...] - tsp_ref[...]
    arg = jnp.concatenate(
        [delta * tw_ref[...] + tph_ref[...],
         age_ref[...] * aw_ref[...] + aph_ref[...]], axis=1)
    feats = _sin(arg)

    acc = jnp.dot(wt_ref[...].astype(jnp.bfloat16), ww_ref[...],
                  preferred_element_type=jnp.float32)
    acc = acc + jnp.dot(feats.astype(jnp.bfloat16), wta_ref[...],
                        preferred_element_type=jnp.float32)
    x = jnp.tanh(acc + b_ref[...])

    iota = lax.broadcasted_iota(jnp.int32, (1, ncat), 1)
    oh = ((tt_ref[...] == iota) | ((vs_ref[...] + ntt) == iota)
          | ((vo_ref[...] + ntt + nvs) == iota)).astype(jnp.bfloat16)
    small = jnp.dot(oh, cat_ref[...], preferred_element_type=jnp.float32)

    emb = x + small
    mean = jnp.mean(emb, axis=1, keepdims=True)
    cen = emb - mean
    var = jnp.mean(cen * cen, axis=1, keepdims=True)
    o_ref[...] = cen * lax.rsqrt(var + eps) * g_ref[...] + bb_ref[...]


def kernel(input_ids, time_stamps, ages, token_type_ids_batch, visit_orders,
           visit_segments, word_table, token_type_table, visit_order_table,
           visit_segment_table, time_w, time_phi, age_w, age_phi, lin_w,
           lin_b, ln_g, ln_b):
    b, s = input_ids.shape
    h = word_table.shape[1]
    t = time_w.shape[1]
    n = b * s
    ntt = token_type_table.shape[0]
    nvs = visit_segment_table.shape[0]
    ncat = ntt + nvs + visit_order_table.shape[0]
    blk = 512

    ids = input_ids.reshape(-1).astype(jnp.int32)
    wt_rows = _sc_gather(word_table, ids)

    ts = time_stamps.reshape(n, 1)
    ts_prev = jnp.concatenate(
        [time_stamps[:, :1], time_stamps[:, :-1]], axis=1).reshape(n, 1)
    ages_r = ages.reshape(n, 1)
    tt = token_type_ids_batch.reshape(n, 1).astype(jnp.int32)
    vs = visit_segments.reshape(n, 1).astype(jnp.int32)
    vo = visit_orders.reshape(n, 1).astype(jnp.int32)

    w_t = lin_w.T
    ww = w_t[:h].astype(jnp.bfloat16)
    wta = w_t[h:].astype(jnp.bfloat16)
    cat_table = jnp.concatenate(
        [token_type_table, visit_segment_table,
         visit_order_table], 0).astype(jnp.bfloat16)

    rep = lambda j: (0, 0)
    shared = (ww, wta, time_w, time_phi, age_w, age_phi, cat_table,
              lin_b.reshape(1, h), ln_g.reshape(1, h), ln_b.reshape(1, h))
    shared_specs = [
        pl.BlockSpec((h, h), rep),
        pl.BlockSpec((2 * t, h), rep),
        pl.BlockSpec((1, t), rep),
        pl.BlockSpec((1, t), rep),
        pl.BlockSpec((1, t), rep),
        pl.BlockSpec((1, t), rep),
        pl.BlockSpec((ncat, h), rep),
        pl.BlockSpec((1, h), rep),
        pl.BlockSpec((1, h), rep),
        pl.BlockSpec((1, h), rep),
    ]

    body = functools.partial(
        _tc_body, ntt=ntt, nvs=nvs, ncat=ncat, eps=1e-12)
    row0 = lambda j: (j, 0)
    chunk_specs = [
        pl.BlockSpec((blk, h), row0),
        pl.BlockSpec((blk, 1), row0),
        pl.BlockSpec((blk, 1), row0),
        pl.BlockSpec((blk, 1), row0),
        pl.BlockSpec((blk, 1), row0),
        pl.BlockSpec((blk, 1), row0),
        pl.BlockSpec((blk, 1), row0),
    ]

    npipe = 2
    per = n // npipe
    wt_chunks = [_sc_gather(word_table, ids[c * per:(c + 1) * per])
                 for c in range(npipe)]
    buf = None
    for c in range(npipe):
        sl = slice(c * per, (c + 1) * per)
        row_c = lambda j, c=c: (j + c * (per // blk), 0)
        chunk_args = (wt_chunks[c], ts[sl], ts_prev[sl], ages_r[sl],
                      tt[sl], vs[sl], vo[sl])
        if buf is None:
            in_specs = chunk_specs + shared_specs
            args = chunk_args + shared
            aliases = {}
        else:
            in_specs = [pl.BlockSpec(memory_space=pl.ANY)] \
                + chunk_specs + shared_specs
            args = (buf,) + chunk_args + shared
            aliases = {0: 0}
        buf = pl.pallas_call(
            body,
            grid=(per // blk,),
            in_specs=in_specs,
            out_specs=pl.BlockSpec((blk, h), row_c),
            out_shape=jax.ShapeDtypeStruct((n, h), jnp.float32),
            input_output_aliases=aliases,
            compiler_params=pltpu.CompilerParams(
                dimension_semantics=("arbitrary",)),
        )(*args)

    return buf.reshape(b, s, h)


# speedup vs baseline: 1.1420x; 1.1420x over previous
import functools

import jax
import jax.numpy as jnp
from jax import lax
from jax.experimental import pallas as pl
from jax.experimental.pallas import tpu as pltpu
from jax.experimental.pallas import tpu_sc as plsc

_SC_CHUNK = 128


def _sin(x):
    ni = jnp.round(x * 0.3183098861837907).astype(jnp.int32)
    n = ni.astype(jnp.float32)
    r = x - n * 3.140625
    r = r - n * 9.676535897932e-4
    r2 = r * r
    p = r * (1.0 + r2 * (-0.16666666666 + r2 * (8.3333333333e-3 + r2 * (
        -1.98412698e-4 + r2 * (2.75573192e-6 + r2 * -2.50521084e-8)))))
    return jnp.where((ni & 1) == 1, -p, p)


def _sc_gather(word_table, ids):
    n = ids.shape[0]
    h = word_table.shape[1]
    info = plsc.get_sparse_core_info()
    nw = info.num_cores * info.num_subcores
    per = n // nw
    ch = min(64, per)
    nch = per // ch

    mesh = plsc.VectorSubcoreMesh(core_axis_name="c", subcore_axis_name="s")

    @functools.partial(
        pl.kernel,
        mesh=mesh,
        out_type=jax.ShapeDtypeStruct((n, h), jnp.float32),
        scratch_types=[
            pltpu.VMEM((ch,), jnp.int32),
            pltpu.VMEM((ch,), jnp.int32),
            pltpu.VMEM((ch, h), jnp.float32),
            pltpu.VMEM((ch, h), jnp.float32),
            pltpu.SemaphoreType.DMA,
            pltpu.SemaphoreType.DMA,
            pltpu.SemaphoreType.DMA,
            pltpu.SemaphoreType.DMA,
        ],
    )
    def gather_kernel(wt_hbm, ids_hbm, out_w, idx0, idx1, rows0, rows1,
                      gsem0, gsem1, wsem0, wsem1):
        wid = lax.axis_index("s") * info.num_cores + lax.axis_index("c")
        base = wid * per
        idx = (idx0, idx1)
        rows = (rows0, rows1)
        gsem = (gsem0, gsem1)
        wsem = (wsem0, wsem1)
        wb = [None, None]
        gather = [None, None]
        for c in range(nch):
            p = c & 1
            off = base + c * ch
            if wb[p] is not None:
                wb[p].wait()
                wb[p] = None
            pltpu.sync_copy(ids_hbm.at[pl.ds(off, ch)], idx[p])
            gather[p] = pltpu.async_copy(wt_hbm.at[idx[p]], rows[p], gsem[p])
            if c >= 1:
                q = (c - 1) & 1
                gather[q].wait()
                wb[q] = pltpu.async_copy(
                    rows[q], out_w.at[pl.ds(base + (c - 1) * ch, ch)], wsem[q])
        last = nch - 1
        p = last & 1
        gather[p].wait()
        pltpu.sync_copy(rows[p], out_w.at[pl.ds(base + last * ch, ch)])
        if nch >= 2 and wb[(last - 1) & 1] is not None:
            wb[(last - 1) & 1].wait()

    return gather_kernel(word_table, ids)


def _tc_body(wt_ref, ts_ref, tsp_ref, age_ref, tt_ref, vs_ref, vo_ref,
             ww_ref, wta_ref, tw_ref, tph_ref, aw_ref, aph_ref,
             cat_ref, b_ref, g_ref, bb_ref, o_ref, *, ntt, nvs, ncat, eps):
    acc = jnp.dot(wt_ref[...].astype(jnp.bfloat16), ww_ref[...],
                  preferred_element_type=jnp.float32)

    delta = ts_ref[...] - tsp_ref[...]
    arg = jnp.concatenate(
        [delta * tw_ref[...] + tph_ref[...],
         age_ref[...] * aw_ref[...] + aph_ref[...]], axis=1)
    feats = _sin(arg)
    acc = acc + jnp.dot(feats.astype(jnp.bfloat16), wta_ref[...],
                        preferred_element_type=jnp.float32)
    x = jnp.tanh(acc + b_ref[...])

    iota = lax.broadcasted_iota(jnp.int32, (1, ncat), 1)
    oh = ((tt_ref[...] == iota) | ((vs_ref[...] + ntt) == iota)
          | ((vo_ref[...] + ntt + nvs) == iota)).astype(jnp.bfloat16)
    small = jnp.dot(oh, cat_ref[...], preferred_element_type=jnp.float32)

    emb = x + small
    mean = jnp.mean(emb, axis=1, keepdims=True)
    cen = emb - mean
    var = jnp.mean(cen * cen, axis=1, keepdims=True)
    o_ref[...] = cen * lax.rsqrt(var + eps) * g_ref[...] + bb_ref[...]


def kernel(input_ids, time_stamps, ages, token_type_ids_batch, visit_orders,
           visit_segments, word_table, token_type_table, visit_order_table,
           visit_segment_table, time_w, time_phi, age_w, age_phi, lin_w,
           lin_b, ln_g, ln_b):
    b, s = input_ids.shape
    h = word_table.shape[1]
    t = time_w.shape[1]
    n = b * s
    ntt = token_type_table.shape[0]
    nvs = visit_segment_table.shape[0]
    ncat = ntt + nvs + visit_order_table.shape[0]
    blk = 512

    ids = input_ids.reshape(-1).astype(jnp.int32)
    wt_rows = _sc_gather(word_table, ids)

    ts = time_stamps.reshape(n, 1)
    ts_prev = jnp.concatenate(
        [time_stamps[:, :1], time_stamps[:, :-1]], axis=1).reshape(n, 1)
    ages_r = ages.reshape(n, 1)
    tt = token_type_ids_batch.reshape(n, 1).astype(jnp.int32)
    vs = visit_segments.reshape(n, 1).astype(jnp.int32)
    vo = visit_orders.reshape(n, 1).astype(jnp.int32)

    w_t = lin_w.T
    ww = w_t[:h].astype(jnp.bfloat16)
    wta = w_t[h:].astype(jnp.bfloat16)
    cat_table = jnp.concatenate(
        [token_type_table, visit_segment_table,
         visit_order_table], 0).astype(jnp.bfloat16)

    rep = lambda j: (0, 0)
    shared = (ww, wta, time_w, time_phi, age_w, age_phi, cat_table,
              lin_b.reshape(1, h), ln_g.reshape(1, h), ln_b.reshape(1, h))
    shared_specs = [
        pl.BlockSpec((h, h), rep),
        pl.BlockSpec((2 * t, h), rep),
        pl.BlockSpec((1, t), rep),
        pl.BlockSpec((1, t), rep),
        pl.BlockSpec((1, t), rep),
        pl.BlockSpec((1, t), rep),
        pl.BlockSpec((ncat, h), rep),
        pl.BlockSpec((1, h), rep),
        pl.BlockSpec((1, h), rep),
        pl.BlockSpec((1, h), rep),
    ]

    body = functools.partial(
        _tc_body, ntt=ntt, nvs=nvs, ncat=ncat, eps=1e-12)
    row0 = lambda j: (j, 0)
    chunk_specs = [
        pl.BlockSpec((blk, h), row0),
        pl.BlockSpec((blk, 1), row0),
        pl.BlockSpec((blk, 1), row0),
        pl.BlockSpec((blk, 1), row0),
        pl.BlockSpec((blk, 1), row0),
        pl.BlockSpec((blk, 1), row0),
        pl.BlockSpec((blk, 1), row0),
    ]
    out = pl.pallas_call(
        body,
        grid=(n // blk,),
        in_specs=chunk_specs + shared_specs,
        out_specs=pl.BlockSpec((blk, h), row0),
        out_shape=jax.ShapeDtypeStruct((n, h), jnp.float32),
        compiler_params=pltpu.CompilerParams(
            dimension_semantics=("arbitrary",)),
    )(wt_rows, ts, ts_prev, ages_r, tt, vs, vo, *shared)

    return out.reshape(b, s, h)


# speedup vs baseline: 1.1961x; 1.0473x over previous
import functools

import jax
import jax.numpy as jnp
from jax import lax
from jax.experimental import pallas as pl
from jax.experimental.pallas import tpu as pltpu
from jax.experimental.pallas import tpu_sc as plsc

_SC_CHUNK = 128


def _sin(x):
    ni = jnp.round(x * 0.3183098861837907).astype(jnp.int32)
    n = ni.astype(jnp.float32)
    r = x - n * 3.140625
    r = r - n * 9.676535897932e-4
    r2 = r * r
    p = r * (1.0 + r2 * (-0.16666666666 + r2 * (8.3333333333e-3 + r2 * (
        -1.98412698e-4 + r2 * (2.75573192e-6 + r2 * -2.50521084e-8)))))
    return jnp.where((ni & 1) == 1, -p, p)


def _sc_gather(word_table, ids):
    n = ids.shape[0]
    h = word_table.shape[1]
    info = plsc.get_sparse_core_info()
    nw = info.num_cores * info.num_subcores
    per = n // nw
    ch = min(_SC_CHUNK, per)
    nch = per // ch

    mesh = plsc.VectorSubcoreMesh(core_axis_name="c", subcore_axis_name="s")

    @functools.partial(
        pl.kernel,
        mesh=mesh,
        out_type=jax.ShapeDtypeStruct((n, h), jnp.float32),
        scratch_types=[
            pltpu.VMEM((ch,), jnp.int32),
            pltpu.VMEM((ch, h), jnp.float32),
            pltpu.SemaphoreType.DMA,
        ],
    )
    def gather_kernel(wt_hbm, ids_hbm, out_w, idx_v, rows_v, sem):
        wid = lax.axis_index("s") * info.num_cores + lax.axis_index("c")
        base = wid * per
        for c in range(nch):
            off = base + c * ch
            pltpu.sync_copy(ids_hbm.at[pl.ds(off, ch)], idx_v)
            pltpu.async_copy(wt_hbm.at[idx_v], rows_v, sem).wait()
            pltpu.sync_copy(rows_v, out_w.at[pl.ds(off, ch)])

    return gather_kernel(word_table, ids)


def _tc_body(wt_ref, ts_ref, tsm_ref, age_ref, tt_ref, vs_ref, vo_ref,
             w_ref, tw_ref, tph_ref, aw_ref, aph_ref,
             cat_ref, b_ref, g_ref, bb_ref, o_ref, *,
             ntt, nvs, ncat, eps, h, bpr):
    dnums = (((1,), (1,)), ((), ()))
    acc = lax.dot_general(wt_ref[...].astype(jnp.bfloat16), w_ref[:, :h],
                          dnums, preferred_element_type=jnp.float32)

    ts = ts_ref[...]
    prev = jnp.concatenate([tsm_ref[-1:], ts[:-1]], axis=0)
    delta = ts - prev
    first = lax.broadcasted_iota(jnp.int32, delta.shape, 0) == 0
    at_row_start = pl.program_id(0) % bpr == 0
    delta = jnp.where(jnp.logical_and(at_row_start, first), 0.0, delta)

    arg = jnp.concatenate(
        [delta * tw_ref[...] + tph_ref[...],
         age_ref[...] * aw_ref[...] + aph_ref[...]], axis=1)
    feats = _sin(arg)
    acc = acc + lax.dot_general(feats.astype(jnp.bfloat16), w_ref[:, h:],
                                dnums, preferred_element_type=jnp.float32)
    x = jnp.tanh(acc + b_ref[...])

    iota = lax.broadcasted_iota(jnp.int32, (1, ncat), 1)
    oh = ((tt_ref[...] == iota) | ((vs_ref[...] + ntt) == iota)
          | ((vo_ref[...] + ntt + nvs) == iota)).astype(jnp.bfloat16)
    small = jnp.dot(oh, cat_ref[...], preferred_element_type=jnp.float32)

    emb = x + small
    mean = jnp.mean(emb, axis=1, keepdims=True)
    cen = emb - mean
    var = jnp.mean(cen * cen, axis=1, keepdims=True)
    o_ref[...] = cen * lax.rsqrt(var + eps) * g_ref[...] + bb_ref[...]


def kernel(input_ids, time_stamps, ages, token_type_ids_batch, visit_orders,
           visit_segments, word_table, token_type_table, visit_order_table,
           visit_segment_table, time_w, time_phi, age_w, age_phi, lin_w,
           lin_b, ln_g, ln_b):
    b, s = input_ids.shape
    h = word_table.shape[1]
    t = time_w.shape[1]
    n = b * s
    ntt = token_type_table.shape[0]
    nvs = visit_segment_table.shape[0]
    ncat = ntt + nvs + visit_order_table.shape[0]
    blk = 512

    ids = input_ids.reshape(-1).astype(jnp.int32)
    wt_rows = _sc_gather(word_table, ids)

    ts = time_stamps.reshape(n, 1)
    ages_r = ages.reshape(n, 1)
    tt = token_type_ids_batch.reshape(n, 1).astype(jnp.int32)
    vs = visit_segments.reshape(n, 1).astype(jnp.int32)
    vo = visit_orders.reshape(n, 1).astype(jnp.int32)

    w_bf = lin_w.astype(jnp.bfloat16)
    cat_table = jnp.concatenate(
        [token_type_table, visit_segment_table,
         visit_order_table], 0).astype(jnp.bfloat16)

    rep = lambda j: (0, 0)
    shared = (w_bf, time_w, time_phi, age_w, age_phi, cat_table,
              lin_b.reshape(1, h), ln_g.reshape(1, h), ln_b.reshape(1, h))
    shared_specs = [
        pl.BlockSpec((h, h + 2 * t), rep),
        pl.BlockSpec((1, t), rep),
        pl.BlockSpec((1, t), rep),
        pl.BlockSpec((1, t), rep),
        pl.BlockSpec((1, t), rep),
        pl.BlockSpec((ncat, h), rep),
        pl.BlockSpec((1, h), rep),
        pl.BlockSpec((1, h), rep),
        pl.BlockSpec((1, h), rep),
    ]

    body = functools.partial(
        _tc_body, ntt=ntt, nvs=nvs, ncat=ncat, eps=1e-12,
        h=h, bpr=s // blk)
    row0 = lambda j: (j, 0)
    rowm = lambda j: (jnp.maximum(j - 1, 0), 0)
    chunk_specs = [
        pl.BlockSpec((blk, h), row0),
        pl.BlockSpec((blk, 1), row0),
        pl.BlockSpec((blk, 1), rowm),
        pl.BlockSpec((blk, 1), row0),
        pl.BlockSpec((blk, 1), row0),
        pl.BlockSpec((blk, 1), row0),
        pl.BlockSpec((blk, 1), row0),
    ]
    out = pl.pallas_call(
        body,
        grid=(n // blk,),
        in_specs=chunk_specs + shared_specs,
        out_specs=pl.BlockSpec((blk, h), row0),
        out_shape=jax.ShapeDtypeStruct((n, h), jnp.float32),
        compiler_params=pltpu.CompilerParams(
            dimension_semantics=("arbitrary",)),
    )(wt_rows, ts, ts, ages_r, tt, vs, vo, *shared)

    return out.reshape(b, s, h)


# speedup vs baseline: 1.2270x; 1.0259x over previous
import functools

import jax
import jax.numpy as jnp
from jax import lax
from jax.experimental import pallas as pl
from jax.experimental.pallas import tpu as pltpu
from jax.experimental.pallas import tpu_sc as plsc

_SC_CHUNK = 128


def _sin(x):
    ni = jnp.round(x * 0.3183098861837907).astype(jnp.int32)
    n = ni.astype(jnp.float32)
    r = x - n * 3.140625
    r = r - n * 9.676535897932e-4
    r2 = r * r
    p = r * (1.0 + r2 * (-0.16666666666 + r2 * (8.3333333333e-3 + r2 * (
        -1.98412698e-4 + r2 * (2.75573192e-6 + r2 * -2.50521084e-8)))))
    return jnp.where((ni & 1) == 1, -p, p)


def _sc_gather(word_table, ids):
    n = ids.shape[0]
    h = word_table.shape[1]
    info = plsc.get_sparse_core_info()
    nw = info.num_cores * info.num_subcores
    per = n // nw
    ch = min(_SC_CHUNK, per)
    nch = per // ch

    mesh = plsc.VectorSubcoreMesh(core_axis_name="c", subcore_axis_name="s")

    @functools.partial(
        pl.kernel,
        mesh=mesh,
        out_type=jax.ShapeDtypeStruct((n, h), jnp.float32),
        scratch_types=[
            pltpu.VMEM((ch,), jnp.int32),
            pltpu.VMEM((ch, h), jnp.float32),
            pltpu.SemaphoreType.DMA,
        ],
    )
    def gather_kernel(wt_hbm, ids_hbm, out_w, idx_v, rows_v, sem):
        wid = lax.axis_index("s") * info.num_cores + lax.axis_index("c")
        base = wid * per
        for c in range(nch):
            off = base + c * ch
            pltpu.sync_copy(ids_hbm.at[pl.ds(off, ch)], idx_v)
            pltpu.async_copy(wt_hbm.at[idx_v], rows_v, sem).wait()
            pltpu.sync_copy(rows_v, out_w.at[pl.ds(off, ch)])

    return gather_kernel(word_table, ids)


def _tc_body(wt_ref, ts_ref, tsm_ref, age_ref, tt_ref, vs_ref, vo_ref,
             w_ref, tw_ref, tph_ref, aw_ref, aph_ref,
             cat_ref, b_ref, g_ref, bb_ref, o_ref, *,
             ntt, nvs, ncat, eps, h, bpr):
    dnums = (((1,), (1,)), ((), ()))
    acc = lax.dot_general(wt_ref[...].astype(jnp.bfloat16), w_ref[:, :h],
                          dnums, preferred_element_type=jnp.float32)

    ts = ts_ref[...]
    prev = jnp.concatenate([tsm_ref[-1:], ts[:-1]], axis=0)
    delta = ts - prev
    first = lax.broadcasted_iota(jnp.int32, delta.shape, 0) == 0
    at_row_start = pl.program_id(0) % bpr == 0
    delta = jnp.where(jnp.logical_and(at_row_start, first), 0.0, delta)

    arg = jnp.concatenate(
        [delta * tw_ref[...] + tph_ref[...],
         age_ref[...] * aw_ref[...] + aph_ref[...]], axis=1)
    feats = _sin(arg)
    acc = acc + lax.dot_general(feats.astype(jnp.bfloat16), w_ref[:, h:],
                                dnums, preferred_element_type=jnp.float32)
    x = jnp.tanh(acc + b_ref[...])

    iota = lax.broadcasted_iota(jnp.int32, (1, ncat), 1)
    oh = ((tt_ref[...] == iota) | ((vs_ref[...] + ntt) == iota)
          | ((vo_ref[...] + ntt + nvs) == iota)).astype(jnp.bfloat16)
    small = jnp.dot(oh, cat_ref[...], preferred_element_type=jnp.float32)

    emb = x + small
    mean = jnp.mean(emb, axis=1, keepdims=True)
    cen = emb - mean
    var = jnp.mean(cen * cen, axis=1, keepdims=True)
    o_ref[...] = cen * lax.rsqrt(var + eps) * g_ref[...] + bb_ref[...]


def kernel(input_ids, time_stamps, ages, token_type_ids_batch, visit_orders,
           visit_segments, word_table, token_type_table, visit_order_table,
           visit_segment_table, time_w, time_phi, age_w, age_phi, lin_w,
           lin_b, ln_g, ln_b):
    b, s = input_ids.shape
    h = word_table.shape[1]
    t = time_w.shape[1]
    n = b * s
    ntt = token_type_table.shape[0]
    nvs = visit_segment_table.shape[0]
    ncat = ntt + nvs + visit_order_table.shape[0]
    blk = 1024

    ids = input_ids.reshape(-1).astype(jnp.int32)
    wt_rows = _sc_gather(word_table, ids)

    ts = time_stamps.reshape(n, 1)
    ages_r = ages.reshape(n, 1)
    tt = token_type_ids_batch.reshape(n, 1).astype(jnp.int32)
    vs = visit_segments.reshape(n, 1).astype(jnp.int32)
    vo = visit_orders.reshape(n, 1).astype(jnp.int32)

    w_bf = lin_w.astype(jnp.bfloat16)
    cat_table = jnp.concatenate(
        [token_type_table, visit_segment_table,
         visit_order_table], 0).astype(jnp.bfloat16)

    rep = lambda j: (0, 0)
    shared = (w_bf, time_w, time_phi, age_w, age_phi, cat_table,
              lin_b.reshape(1, h), ln_g.reshape(1, h), ln_b.reshape(1, h))
    shared_specs = [
        pl.BlockSpec((h, h + 2 * t), rep),
        pl.BlockSpec((1, t), rep),
        pl.BlockSpec((1, t), rep),
        pl.BlockSpec((1, t), rep),
        pl.BlockSpec((1, t), rep),
        pl.BlockSpec((ncat, h), rep),
        pl.BlockSpec((1, h), rep),
        pl.BlockSpec((1, h), rep),
        pl.BlockSpec((1, h), rep),
    ]

    body = functools.partial(
        _tc_body, ntt=ntt, nvs=nvs, ncat=ncat, eps=1e-12,
        h=h, bpr=s // blk)
    row0 = lambda j: (j, 0)
    rowm = lambda j: (jnp.maximum(j - 1, 0), 0)
    chunk_specs = [
        pl.BlockSpec((blk, h), row0),
        pl.BlockSpec((blk, 1), row0),
        pl.BlockSpec((blk, 1), rowm),
        pl.BlockSpec((blk, 1), row0),
        pl.BlockSpec((blk, 1), row0),
        pl.BlockSpec((blk, 1), row0),
        pl.BlockSpec((blk, 1), row0),
    ]
    out = pl.pallas_call(
        body,
        grid=(n // blk,),
        in_specs=chunk_specs + shared_specs,
        out_specs=pl.BlockSpec((blk, h), row0),
        out_shape=jax.ShapeDtypeStruct((n, h), jnp.float32),
        compiler_params=pltpu.CompilerParams(
            dimension_semantics=("arbitrary",)),
    )(wt_rows, ts, ts, ages_r, tt, vs, vo, *shared)

    return out.reshape(b, s, h)
